# Initial kernel scaffold; baseline (speedup 1.0000x reference)
import jax
import jax.numpy as jnp
from jax.experimental import pallas as pl


def kernel(x, pos, batch, idx, params):
    raise NotImplementedError("write your pallas kernel here")


# speedup vs baseline: 3.2362x; 3.2362x over previous
import functools

import jax
import jax.numpy as jnp
from jax import lax
from jax.experimental import pallas as pl
from jax.experimental.pallas import tpu as pltpu
from jax.experimental.pallas import tpu_sc as plsc

_N = 10000
_NPAD = 10240
_S = 2048
_INC = 128
_K = 64
_SB = 256
_GRID = _S // _SB
_EGRID = 32
_DTAB = 256
_RADII = (0.1, 0.2, 0.4)
_KS = (16, 32, 64)
_DIMS = ((32, 32, 64), (64, 64, 128), (64, 96, 128))
_EPS = 1e-5


def _topk_body(posT_ref, pos_s_ref, topi_ref, topd_ref, d2_ref):
    ps = pos_s_ref[...]
    d2 = jnp.zeros((_SB, _NPAD), jnp.float32)
    for d in range(3):
        diff = ps[:, d:d + 1] - posT_ref[d:d + 1, :]
        d2 = d2 + diff * diff
    d2_ref[...] = d2
    lane = lax.broadcasted_iota(jnp.int32, (_SB, _NPAD), 1)
    col = lax.broadcasted_iota(jnp.int32, (_SB, _K), 1)

    def step(t, carry):
        ti, td = carry
        cur = d2_ref[...]
        m = jnp.min(cur, axis=1, keepdims=True)
        am = jnp.min(jnp.where(cur == m, lane, jnp.int32(2 ** 30)),
                     axis=1, keepdims=True)
        d2_ref[...] = jnp.where(lane == am, jnp.float32(jnp.inf), cur)
        ti = jnp.where(col == t, am, ti)
        td = jnp.where(col == t, m, td)
        return ti, td

    ti, td = lax.fori_loop(
        0, _K, step,
        (jnp.zeros((_SB, _K), jnp.int32), jnp.zeros((_SB, _K), jnp.float32)))
    topi_ref[...] = ti
    topd_ref[...] = td


def _run_topk(posT, pos_s):
    return pl.pallas_call(
        _topk_body,
        grid=(_GRID,),
        in_specs=[
            pl.BlockSpec((3, _NPAD), lambda i: (0, 0)),
            pl.BlockSpec((_SB, 3), lambda i: (i, 0)),
        ],
        out_specs=[
            pl.BlockSpec((_SB, _K), lambda i: (i, 0)),
            pl.BlockSpec((_SB, _K), lambda i: (i, 0)),
        ],
        out_shape=[
            jax.ShapeDtypeStruct((_S, _K), jnp.int32),
            jax.ShapeDtypeStruct((_S, _K), jnp.float32),
        ],
        scratch_shapes=[pltpu.VMEM((_SB, _NPAD), jnp.float32)],
    )(posT, pos_s)


def _sc_gather(table, idx_flat):
    info = plsc.get_sparse_core_info()
    nw = info.num_cores * info.num_subcores
    b = idx_flat.shape[0]
    d = table.shape[1]
    bpw = b // nw
    cs = 256
    nch = bpw // cs
    mesh = plsc.VectorSubcoreMesh(core_axis_name="c", subcore_axis_name="s")

    @functools.partial(
        pl.kernel,
        mesh=mesh,
        out_type=jax.ShapeDtypeStruct((b, d), jnp.float32),
        scratch_types=[
            pltpu.VMEM((cs,), jnp.int32),
            pltpu.VMEM((cs, d), jnp.float32),
            pltpu.SemaphoreType.DMA,
        ],
    )
    def gather_kernel(table_hbm, idx_hbm, out_hbm, idx_v, rows_v, sem):
        wid = lax.axis_index("s") * info.num_cores + lax.axis_index("c")
        base = wid * bpw
        for j in range(nch):
            off = base + j * cs
            pltpu.sync_copy(idx_hbm.at[pl.ds(off, cs)], idx_v)
            pltpu.async_copy(table_hbm.at[idx_v], rows_v, sem).wait()
            pltpu.sync_copy(rows_v, out_hbm.at[pl.ds(off, cs)])

    return gather_kernel(table, idx_flat)


def _partial_rows(h, m, c):
    ps = jnp.sum(h * m, axis=0, keepdims=True)
    pq = jnp.sum(h * h * m, axis=0, keepdims=True)
    cnt = jnp.sum(m)
    lane = lax.broadcasted_iota(jnp.int32, (1, c), 1)
    pc = jnp.where(lane == 0, cnt, jnp.float32(0.0))
    return jnp.concatenate([ps, pq, pc, jnp.zeros((5, c), jnp.float32)], axis=0)


def _reduce_stats(part_ref):
    tot = part_ref[0]
    for g in range(1, part_ref.shape[0]):
        tot = tot + part_ref[g]
    cnt = jnp.maximum(jnp.sum(tot[2:3, 0:1]), 1.0)
    mean = tot[0:1, :] / cnt
    var = tot[1:2, :] / cnt - mean * mean
    return mean, var


def _layer1_body(r2, c1, xg_ref, aux_ref, w1xt_ref, w1pt_ref, p1_ref,
                 h_ref, part_ref):
    xg = xg_ref[...]
    aux = aux_ref[...]
    xp = xg[:, 0:_INC]
    rel = xg[:, _INC:_INC + 3] - aux[:, 0:3]
    h = (jnp.dot(xp, w1xt_ref[...], preferred_element_type=jnp.float32)
         + jnp.dot(rel, w1pt_ref[...], preferred_element_type=jnp.float32)
         + p1_ref[0:1, :])
    m = (aux[:, 3:4] <= r2).astype(jnp.float32)
    h_ref[...] = h
    part_ref[0] = _partial_rows(h, m, c1)


def _run_layer1(r2, c1, k, xg_s, aux, w1xt, w1pt, p1):
    e = _S * k
    eb = e // _EGRID
    return pl.pallas_call(
        functools.partial(_layer1_body, r2, c1),
        grid=(_EGRID,),
        in_specs=[
            pl.BlockSpec((eb, _DTAB), lambda i: (i, 0)),
            pl.BlockSpec((eb, 4), lambda i: (i, 0)),
            pl.BlockSpec((_INC, c1), lambda i: (0, 0)),
            pl.BlockSpec((3, c1), lambda i: (0, 0)),
            pl.BlockSpec((8, c1), lambda i: (0, 0)),
        ],
        out_specs=[
            pl.BlockSpec((eb, c1), lambda i: (i, 0)),
            pl.BlockSpec((1, 8, c1), lambda i: (i, 0, 0)),
        ],
        out_shape=[
            jax.ShapeDtypeStruct((e, c1), jnp.float32),
            jax.ShapeDtypeStruct((_EGRID, 8, c1), jnp.float32),
        ],
    )(xg_s, aux, w1xt, w1pt, p1)


def _mid_body(r2, cout, h_ref, aux_ref, part_in_ref, pcur_ref, wt_ref,
              pnext_ref, hout_ref, part_ref):
    mean, var = _reduce_stats(part_in_ref)
    gamma = pcur_ref[1:2, :]
    beta = pcur_ref[2:3, :]
    h = h_ref[...]
    hn = gamma * (h - mean) * lax.rsqrt(var + _EPS) + beta
    hn = jnp.maximum(hn, 0.0)
    h2 = (jnp.dot(hn, wt_ref[...], preferred_element_type=jnp.float32)
          + pnext_ref[0:1, :])
    m = (aux_ref[:, 3:4] <= r2).astype(jnp.float32)
    hout_ref[...] = h2
    part_ref[0] = _partial_rows(h2, m, cout)


def _run_mid(r2, cin, cout, k, h, aux, part_in, pcur, wt, pnext):
    e = _S * k
    eb = e // _EGRID
    return pl.pallas_call(
        functools.partial(_mid_body, r2, cout),
        grid=(_EGRID,),
        in_specs=[
            pl.BlockSpec((eb, cin), lambda i: (i, 0)),
            pl.BlockSpec((eb, 4), lambda i: (i, 0)),
            pl.BlockSpec((_EGRID, 8, cin), lambda i: (0, 0, 0)),
            pl.BlockSpec((8, cin), lambda i: (0, 0)),
            pl.BlockSpec((cin, cout), lambda i: (0, 0)),
            pl.BlockSpec((8, cout), lambda i: (0, 0)),
        ],
        out_specs=[
            pl.BlockSpec((eb, cout), lambda i: (i, 0)),
            pl.BlockSpec((1, 8, cout), lambda i: (i, 0, 0)),
        ],
        out_shape=[
            jax.ShapeDtypeStruct((e, cout), jnp.float32),
            jax.ShapeDtypeStruct((_EGRID, 8, cout), jnp.float32),
        ],
    )(h, aux, part_in, pcur, wt, pnext)


def _final_body(r2, c3, k, h_ref, topd_ref, part_in_ref, pcur_ref, out_ref):
    mean, var = _reduce_stats(part_in_ref)
    gamma = pcur_ref[1:2, :].reshape(1, 1, c3)
    beta = pcur_ref[2:3, :].reshape(1, 1, c3)
    mean3 = mean.reshape(1, 1, c3)
    scale3 = lax.rsqrt(var + _EPS).reshape(1, 1, c3)
    h = h_ref[...]
    hn = gamma * (h - mean3) * scale3 + beta
    hn = jnp.maximum(hn, 0.0)
    valid = topd_ref[:, 0:k] <= r2
    acc = jnp.full((_SB, c3), -jnp.inf, jnp.float32)
    for j in range(k):
        vj = valid[:, j:j + 1]
        acc = jnp.maximum(acc, jnp.where(vj, hn[:, j, :], -jnp.inf))
    anyv = jnp.max(valid.astype(jnp.float32), axis=1, keepdims=True) > 0.0
    out_ref[...] = jnp.where(anyv, acc, 0.0)


def _run_final(r2, c3, k, h3d, topd, part_in, pcur):
    return pl.pallas_call(
        functools.partial(_final_body, r2, c3, k),
        grid=(_GRID,),
        in_specs=[
            pl.BlockSpec((_SB, k, c3), lambda i: (i, 0, 0)),
            pl.BlockSpec((_SB, _K), lambda i: (i, 0)),
            pl.BlockSpec((_EGRID, 8, c3), lambda i: (0, 0, 0)),
            pl.BlockSpec((8, c3), lambda i: (0, 0)),
        ],
        out_specs=pl.BlockSpec((_SB, c3), lambda i: (i, 0)),
        out_shape=jax.ShapeDtypeStruct((_S, c3), jnp.float32),
    )(h3d, topd, part_in, pcur)


def _pack_params(lyr, c):
    return jnp.concatenate(
        [lyr["b"][None, :], lyr["gamma"][None, :], lyr["beta"][None, :],
         jnp.zeros((5, c), jnp.float32)], axis=0)


def kernel(x, pos, batch, idx, params):
    x = x.astype(jnp.float32)
    pos = pos.astype(jnp.float32)
    pos_s = pos[idx]
    batch_s = batch[idx]
    posT = jnp.full((3, _NPAD), 1e9, jnp.float32).at[:, :_N].set(pos.T)
    topi, topd = _run_topk(posT, pos_s)

    table = jnp.concatenate(
        [x, pos, jnp.zeros((_N, _DTAB - _INC - 3), jnp.float32)], axis=1)
    xg = _sc_gather(table, topi.reshape(_S * _K))
    xg3 = xg.reshape(_S, _K, _DTAB)

    outs = []
    for si in range(3):
        r2 = float(_RADII[si]) * float(_RADII[si])
        k = _KS[si]
        c1, c2, c3 = _DIMS[si]
        layers = params[si]
        e = _S * k
        xg_s = xg3[:, :k, :].reshape(e, _DTAB)
        aux = jnp.concatenate(
            [jnp.repeat(pos_s, k, axis=0), topd[:, :k].reshape(e, 1)], axis=1)
        w1 = layers[0]["W"]
        w1xt = w1[:, :_INC].T
        w1pt = w1[:, _INC:_INC + 3].T
        p1 = _pack_params(layers[0], c1)
        p2 = _pack_params(layers[1], c2)
        p3 = _pack_params(layers[2], c3)
        h1, part1 = _run_layer1(r2, c1, k, xg_s, aux, w1xt, w1pt, p1)
        h2, part2 = _run_mid(r2, c1, c2, k, h1, aux, part1, p1,
                             layers[1]["W"].T, p2)
        h3, part3 = _run_mid(r2, c2, c3, k, h2, aux, part2, p2,
                             layers[2]["W"].T, p3)
        agg = _run_final(r2, c3, k, h3.reshape(_S, k, c3), topd, part3, p3)
        outs.append(agg)

    x_out = jnp.concatenate(outs, axis=1)
    return x_out, pos_s, batch_s


# speedup vs baseline: 3.3319x; 1.0296x over previous
import functools

import jax
import jax.numpy as jnp
from jax import lax
from jax.experimental import pallas as pl
from jax.experimental.pallas import tpu as pltpu
from jax.experimental.pallas import tpu_sc as plsc

_N = 10000
_NPAD = 10240
_S = 2048
_INC = 128
_K = 64
_SB = 256
_GRID = _S // _SB
_EGRID = 32
_DTAB = 256
_RADII = (0.1, 0.2, 0.4)
_KS = (16, 32, 64)
_DIMS = ((32, 32, 64), (64, 64, 128), (64, 96, 128))
_EPS = 1e-5


def _topk_body(posT_ref, pos_s_ref, topi_ref, topd_ref, d2_ref):
    ps = pos_s_ref[...]
    d2 = jnp.zeros((_SB, _NPAD), jnp.float32)
    for d in range(3):
        diff = ps[:, d:d + 1] - posT_ref[d:d + 1, :]
        d2 = d2 + diff * diff
    d2_ref[...] = d2
    lane = lax.broadcasted_iota(jnp.int32, (_SB, _NPAD), 1)
    col = lax.broadcasted_iota(jnp.int32, (_SB, _K), 1)

    def step(t, carry):
        ti, td = carry
        cur = d2_ref[...]
        m = jnp.min(cur, axis=1, keepdims=True)
        eq = cur == m
        am = jnp.min(jnp.where(eq, lane, jnp.int32(2 ** 30)),
                     axis=1, keepdims=True)
        d2_ref[...] = jnp.where(eq, jnp.float32(jnp.inf), cur)
        ti = jnp.where(col == t, am, ti)
        td = jnp.where(col == t, m, td)
        return ti, td

    ti, td = lax.fori_loop(
        0, _K, step,
        (jnp.zeros((_SB, _K), jnp.int32), jnp.zeros((_SB, _K), jnp.float32)))
    topi_ref[...] = ti
    topd_ref[...] = td


def _run_topk(posT, pos_s):
    return pl.pallas_call(
        _topk_body,
        grid=(_GRID,),
        in_specs=[
            pl.BlockSpec((3, _NPAD), lambda i: (0, 0)),
            pl.BlockSpec((_SB, 3), lambda i: (i, 0)),
        ],
        out_specs=[
            pl.BlockSpec((_SB, _K), lambda i: (i, 0)),
            pl.BlockSpec((_SB, _K), lambda i: (i, 0)),
        ],
        out_shape=[
            jax.ShapeDtypeStruct((_S, _K), jnp.int32),
            jax.ShapeDtypeStruct((_S, _K), jnp.float32),
        ],
        scratch_shapes=[pltpu.VMEM((_SB, _NPAD), jnp.float32)],
    )(posT, pos_s)


def _sc_gather(table, idx_flat):
    info = plsc.get_sparse_core_info()
    nw = info.num_cores * info.num_subcores
    b = idx_flat.shape[0]
    d = table.shape[1]
    bpw = b // nw
    cs = 256
    nch = bpw // cs
    mesh = plsc.VectorSubcoreMesh(core_axis_name="c", subcore_axis_name="s")

    @functools.partial(
        pl.kernel,
        mesh=mesh,
        out_type=jax.ShapeDtypeStruct((b, d), jnp.float32),
        scratch_types=[
            pltpu.VMEM((cs,), jnp.int32),
            pltpu.VMEM((cs, d), jnp.float32),
            pltpu.SemaphoreType.DMA,
        ],
    )
    def gather_kernel(table_hbm, idx_hbm, out_hbm, idx_v, rows_v, sem):
        wid = lax.axis_index("s") * info.num_cores + lax.axis_index("c")
        base = wid * bpw
        for j in range(nch):
            off = base + j * cs
            pltpu.sync_copy(idx_hbm.at[pl.ds(off, cs)], idx_v)
            pltpu.async_copy(table_hbm.at[idx_v], rows_v, sem).wait()
            pltpu.sync_copy(rows_v, out_hbm.at[pl.ds(off, cs)])

    return gather_kernel(table, idx_flat)


def _partial_rows(h, m, c):
    ps = jnp.sum(h * m, axis=0, keepdims=True)
    pq = jnp.sum(h * h * m, axis=0, keepdims=True)
    cnt = jnp.sum(m)
    lane = lax.broadcasted_iota(jnp.int32, (1, c), 1)
    pc = jnp.where(lane == 0, cnt, jnp.float32(0.0))
    return jnp.concatenate([ps, pq, pc, jnp.zeros((5, c), jnp.float32)], axis=0)


def _reduce_stats(part_ref):
    tot = part_ref[0]
    for g in range(1, part_ref.shape[0]):
        tot = tot + part_ref[g]
    cnt = jnp.maximum(jnp.sum(tot[2:3, 0:1]), 1.0)
    mean = tot[0:1, :] / cnt
    var = tot[1:2, :] / cnt - mean * mean
    return mean, var


def _layer1_body(r2, c1, xg_ref, aux_ref, w1xt_ref, w1pt_ref, p1_ref,
                 h_ref, part_ref):
    xg = xg_ref[...]
    aux = aux_ref[...]
    xp = xg[:, 0:_INC]
    rel = xg[:, _INC:_INC + 3] - aux[:, 0:3]
    h = (jnp.dot(xp, w1xt_ref[...], preferred_element_type=jnp.float32)
         + jnp.dot(rel, w1pt_ref[...], preferred_element_type=jnp.float32)
         + p1_ref[0:1, :])
    m = (aux[:, 3:4] <= r2).astype(jnp.float32)
    h_ref[...] = h
    part_ref[0] = _partial_rows(h, m, c1)


def _run_layer1(r2, c1, k, xg_s, aux, w1xt, w1pt, p1):
    e = _S * k
    eb = e // _EGRID
    return pl.pallas_call(
        functools.partial(_layer1_body, r2, c1),
        grid=(_EGRID,),
        in_specs=[
            pl.BlockSpec((eb, _DTAB), lambda i: (i, 0)),
            pl.BlockSpec((eb, 4), lambda i: (i, 0)),
            pl.BlockSpec((_INC, c1), lambda i: (0, 0)),
            pl.BlockSpec((3, c1), lambda i: (0, 0)),
            pl.BlockSpec((8, c1), lambda i: (0, 0)),
        ],
        out_specs=[
            pl.BlockSpec((eb, c1), lambda i: (i, 0)),
            pl.BlockSpec((1, 8, c1), lambda i: (i, 0, 0)),
        ],
        out_shape=[
            jax.ShapeDtypeStruct((e, c1), jnp.float32),
            jax.ShapeDtypeStruct((_EGRID, 8, c1), jnp.float32),
        ],
    )(xg_s, aux, w1xt, w1pt, p1)


def _mid_body(r2, cout, h_ref, aux_ref, part_in_ref, pcur_ref, wt_ref,
              pnext_ref, hout_ref, part_ref):
    mean, var = _reduce_stats(part_in_ref)
    gamma = pcur_ref[1:2, :]
    beta = pcur_ref[2:3, :]
    h = h_ref[...]
    hn = gamma * (h - mean) * lax.rsqrt(var + _EPS) + beta
    hn = jnp.maximum(hn, 0.0)
    h2 = (jnp.dot(hn, wt_ref[...], preferred_element_type=jnp.float32)
          + pnext_ref[0:1, :])
    m = (aux_ref[:, 3:4] <= r2).astype(jnp.float32)
    hout_ref[...] = h2
    part_ref[0] = _partial_rows(h2, m, cout)


def _run_mid(r2, cin, cout, k, h, aux, part_in, pcur, wt, pnext):
    e = _S * k
    eb = e // _EGRID
    return pl.pallas_call(
        functools.partial(_mid_body, r2, cout),
        grid=(_EGRID,),
        in_specs=[
            pl.BlockSpec((eb, cin), lambda i: (i, 0)),
            pl.BlockSpec((eb, 4), lambda i: (i, 0)),
            pl.BlockSpec((_EGRID, 8, cin), lambda i: (0, 0, 0)),
            pl.BlockSpec((8, cin), lambda i: (0, 0)),
            pl.BlockSpec((cin, cout), lambda i: (0, 0)),
            pl.BlockSpec((8, cout), lambda i: (0, 0)),
        ],
        out_specs=[
            pl.BlockSpec((eb, cout), lambda i: (i, 0)),
            pl.BlockSpec((1, 8, cout), lambda i: (i, 0, 0)),
        ],
        out_shape=[
            jax.ShapeDtypeStruct((e, cout), jnp.float32),
            jax.ShapeDtypeStruct((_EGRID, 8, cout), jnp.float32),
        ],
    )(h, aux, part_in, pcur, wt, pnext)


def _final_body(r2, c3, k, h_ref, topd_ref, part_in_ref, pcur_ref, out_ref):
    mean, var = _reduce_stats(part_in_ref)
    gamma = pcur_ref[1:2, :].reshape(1, 1, c3)
    beta = pcur_ref[2:3, :].reshape(1, 1, c3)
    mean3 = mean.reshape(1, 1, c3)
    scale3 = lax.rsqrt(var + _EPS).reshape(1, 1, c3)
    h = h_ref[...]
    hn = gamma * (h - mean3) * scale3 + beta
    hn = jnp.maximum(hn, 0.0)
    valid = topd_ref[:, 0:k] <= r2
    acc = jnp.full((_SB, c3), -jnp.inf, jnp.float32)
    for j in range(k):
        vj = valid[:, j:j + 1]
        acc = jnp.maximum(acc, jnp.where(vj, hn[:, j, :], -jnp.inf))
    anyv = jnp.max(valid.astype(jnp.float32), axis=1, keepdims=True) > 0.0
    out_ref[...] = jnp.where(anyv, acc, 0.0)


def _run_final(r2, c3, k, h3d, topd, part_in, pcur):
    return pl.pallas_call(
        functools.partial(_final_body, r2, c3, k),
        grid=(_GRID,),
        in_specs=[
            pl.BlockSpec((_SB, k, c3), lambda i: (i, 0, 0)),
            pl.BlockSpec((_SB, _K), lambda i: (i, 0)),
            pl.BlockSpec((_EGRID, 8, c3), lambda i: (0, 0, 0)),
            pl.BlockSpec((8, c3), lambda i: (0, 0)),
        ],
        out_specs=pl.BlockSpec((_SB, c3), lambda i: (i, 0)),
        out_shape=jax.ShapeDtypeStruct((_S, c3), jnp.float32),
    )(h3d, topd, part_in, pcur)


def _pack_params(lyr, c):
    return jnp.concatenate(
        [lyr["b"][None, :], lyr["gamma"][None, :], lyr["beta"][None, :],
         jnp.zeros((5, c), jnp.float32)], axis=0)


def kernel(x, pos, batch, idx, params):
    x = x.astype(jnp.float32)
    pos = pos.astype(jnp.float32)
    pos_s = pos[idx]
    batch_s = batch[idx]
    posT = jnp.full((3, _NPAD), 1e9, jnp.float32).at[:, :_N].set(pos.T)
    topi, topd = _run_topk(posT, pos_s)

    table = jnp.concatenate(
        [x, pos, jnp.zeros((_N, _DTAB - _INC - 3), jnp.float32)], axis=1)
    xg = _sc_gather(table, topi.reshape(_S * _K))
    xg3 = xg.reshape(_S, _K, _DTAB)

    outs = []
    for si in range(3):
        r2 = float(_RADII[si]) * float(_RADII[si])
        k = _KS[si]
        c1, c2, c3 = _DIMS[si]
        layers = params[si]
        e = _S * k
        xg_s = xg3[:, :k, :].reshape(e, _DTAB)
        aux = jnp.concatenate(
            [jnp.repeat(pos_s, k, axis=0), topd[:, :k].reshape(e, 1)], axis=1)
        w1 = layers[0]["W"]
        w1xt = w1[:, :_INC].T
        w1pt = w1[:, _INC:_INC + 3].T
        p1 = _pack_params(layers[0], c1)
        p2 = _pack_params(layers[1], c2)
        p3 = _pack_params(layers[2], c3)
        h1, part1 = _run_layer1(r2, c1, k, xg_s, aux, w1xt, w1pt, p1)
        h2, part2 = _run_mid(r2, c1, c2, k, h1, aux, part1, p1,
                             layers[1]["W"].T, p2)
        h3, part3 = _run_mid(r2, c2, c3, k, h2, aux, part2, p2,
                             layers[2]["W"].T, p3)
        agg = _run_final(r2, c3, k, h3.reshape(_S, k, c3), topd, part3, p3)
        outs.append(agg)

    x_out = jnp.concatenate(outs, axis=1)
    return x_out, pos_s, batch_s


# speedup vs baseline: 3.6670x; 1.1006x over previous
import functools

import jax
import jax.numpy as jnp
from jax import lax
from jax.experimental import pallas as pl
from jax.experimental.pallas import tpu as pltpu
from jax.experimental.pallas import tpu_sc as plsc

_N = 10000
_NPAD = 10240
_S = 2048
_INC = 128
_K = 64
_SB = 256
_GRID = _S // _SB
_EGRID = 32
_DTAB = 256
_RADII = (0.1, 0.2, 0.4)
_KS = (16, 32, 64)
_DIMS = ((32, 32, 64), (64, 64, 128), (64, 96, 128))
_EPS = 1e-5


def _topk_body(posT_ref, pos_s_ref, topi_ref, topd_ref, d2_ref):
    ps = pos_s_ref[...]
    d2 = jnp.zeros((_SB, _NPAD), jnp.float32)
    for d in range(3):
        diff = ps[:, d:d + 1] - posT_ref[d:d + 1, :]
        d2 = d2 + diff * diff
    d2_ref[...] = d2
    lane = lax.broadcasted_iota(jnp.int32, (_SB, _NPAD), 1)
    col = lax.broadcasted_iota(jnp.int32, (_SB, _K), 1)

    def step(t, carry):
        m, ti, td = carry
        cur = d2_ref[...]
        am = jnp.min(jnp.where(cur == m, lane, jnp.int32(2 ** 30)),
                     axis=1, keepdims=True)
        ti = jnp.where(col == t, am, ti)
        td = jnp.where(col == t, m, td)
        m = jnp.min(jnp.where(cur > m, cur, jnp.float32(jnp.inf)),
                    axis=1, keepdims=True)
        return m, ti, td

    m0 = jnp.min(d2_ref[...], axis=1, keepdims=True)
    _, ti, td = lax.fori_loop(
        0, _K, step,
        (m0, jnp.zeros((_SB, _K), jnp.int32), jnp.zeros((_SB, _K), jnp.float32)))
    topi_ref[...] = ti
    topd_ref[...] = td


def _run_topk(posT, pos_s):
    return pl.pallas_call(
        _topk_body,
        grid=(_GRID,),
        in_specs=[
            pl.BlockSpec((3, _NPAD), lambda i: (0, 0)),
            pl.BlockSpec((_SB, 3), lambda i: (i, 0)),
        ],
        out_specs=[
            pl.BlockSpec((_SB, _K), lambda i: (i, 0)),
            pl.BlockSpec((_SB, _K), lambda i: (i, 0)),
        ],
        out_shape=[
            jax.ShapeDtypeStruct((_S, _K), jnp.int32),
            jax.ShapeDtypeStruct((_S, _K), jnp.float32),
        ],
        scratch_shapes=[pltpu.VMEM((_SB, _NPAD), jnp.float32)],
    )(posT, pos_s)


def _sc_gather(table, idx_flat):
    info = plsc.get_sparse_core_info()
    nw = info.num_cores * info.num_subcores
    b = idx_flat.shape[0]
    d = table.shape[1]
    bpw = b // nw
    cs = 256
    nch = bpw // cs
    mesh = plsc.VectorSubcoreMesh(core_axis_name="c", subcore_axis_name="s")

    @functools.partial(
        pl.kernel,
        mesh=mesh,
        out_type=jax.ShapeDtypeStruct((b, d), jnp.float32),
        scratch_types=[
            pltpu.VMEM((cs,), jnp.int32),
            pltpu.VMEM((cs, d), jnp.float32),
            pltpu.SemaphoreType.DMA,
        ],
    )
    def gather_kernel(table_hbm, idx_hbm, out_hbm, idx_v, rows_v, sem):
        wid = lax.axis_index("s") * info.num_cores + lax.axis_index("c")
        base = wid * bpw
        for j in range(nch):
            off = base + j * cs
            pltpu.sync_copy(idx_hbm.at[pl.ds(off, cs)], idx_v)
            pltpu.async_copy(table_hbm.at[idx_v], rows_v, sem).wait()
            pltpu.sync_copy(rows_v, out_hbm.at[pl.ds(off, cs)])

    return gather_kernel(table, idx_flat)


def _partial_rows(h, m, c):
    ps = jnp.sum(h * m, axis=0, keepdims=True)
    pq = jnp.sum(h * h * m, axis=0, keepdims=True)
    cnt = jnp.sum(m)
    lane = lax.broadcasted_iota(jnp.int32, (1, c), 1)
    pc = jnp.where(lane == 0, cnt, jnp.float32(0.0))
    return jnp.concatenate([ps, pq, pc, jnp.zeros((5, c), jnp.float32)], axis=0)


def _reduce_stats(part_ref):
    tot = part_ref[0]
    for g in range(1, part_ref.shape[0]):
        tot = tot + part_ref[g]
    cnt = jnp.maximum(jnp.sum(tot[2:3, 0:1]), 1.0)
    mean = tot[0:1, :] / cnt
    var = tot[1:2, :] / cnt - mean * mean
    return mean, var


def _layer1_body(r2, c1, xg_ref, aux_ref, w1xt_ref, w1pt_ref, p1_ref,
                 h_ref, part_ref):
    xg = xg_ref[...]
    aux = aux_ref[...]
    xp = xg[:, 0:_INC]
    rel = xg[:, _INC:_INC + 3] - aux[:, 0:3]
    h = (jnp.dot(xp, w1xt_ref[...], preferred_element_type=jnp.float32)
         + jnp.dot(rel, w1pt_ref[...], preferred_element_type=jnp.float32)
         + p1_ref[0:1, :])
    m = (aux[:, 3:4] <= r2).astype(jnp.float32)
    h_ref[...] = h
    part_ref[0] = _partial_rows(h, m, c1)


def _run_layer1(r2, c1, k, xg_s, aux, w1xt, w1pt, p1):
    e = _S * k
    eb = e // _EGRID
    return pl.pallas_call(
        functools.partial(_layer1_body, r2, c1),
        grid=(_EGRID,),
        in_specs=[
            pl.BlockSpec((eb, _DTAB), lambda i: (i, 0)),
            pl.BlockSpec((eb, 4), lambda i: (i, 0)),
            pl.BlockSpec((_INC, c1), lambda i: (0, 0)),
            pl.BlockSpec((3, c1), lambda i: (0, 0)),
            pl.BlockSpec((8, c1), lambda i: (0, 0)),
        ],
        out_specs=[
            pl.BlockSpec((eb, c1), lambda i: (i, 0)),
            pl.BlockSpec((1, 8, c1), lambda i: (i, 0, 0)),
        ],
        out_shape=[
            jax.ShapeDtypeStruct((e, c1), jnp.float32),
            jax.ShapeDtypeStruct((_EGRID, 8, c1), jnp.float32),
        ],
    )(xg_s, aux, w1xt, w1pt, p1)


def _mid_body(r2, cout, h_ref, aux_ref, part_in_ref, pcur_ref, wt_ref,
              pnext_ref, hout_ref, part_ref):
    mean, var = _reduce_stats(part_in_ref)
    gamma = pcur_ref[1:2, :]
    beta = pcur_ref[2:3, :]
    h = h_ref[...]
    hn = gamma * (h - mean) * lax.rsqrt(var + _EPS) + beta
    hn = jnp.maximum(hn, 0.0)
    h2 = (jnp.dot(hn, wt_ref[...], preferred_element_type=jnp.float32)
          + pnext_ref[0:1, :])
    m = (aux_ref[:, 3:4] <= r2).astype(jnp.float32)
    hout_ref[...] = h2
    part_ref[0] = _partial_rows(h2, m, cout)


def _run_mid(r2, cin, cout, k, h, aux, part_in, pcur, wt, pnext):
    e = _S * k
    eb = e // _EGRID
    return pl.pallas_call(
        functools.partial(_mid_body, r2, cout),
        grid=(_EGRID,),
        in_specs=[
            pl.BlockSpec((eb, cin), lambda i: (i, 0)),
            pl.BlockSpec((eb, 4), lambda i: (i, 0)),
            pl.BlockSpec((_EGRID, 8, cin), lambda i: (0, 0, 0)),
            pl.BlockSpec((8, cin), lambda i: (0, 0)),
            pl.BlockSpec((cin, cout), lambda i: (0, 0)),
            pl.BlockSpec((8, cout), lambda i: (0, 0)),
        ],
        out_specs=[
            pl.BlockSpec((eb, cout), lambda i: (i, 0)),
            pl.BlockSpec((1, 8, cout), lambda i: (i, 0, 0)),
        ],
        out_shape=[
            jax.ShapeDtypeStruct((e, cout), jnp.float32),
            jax.ShapeDtypeStruct((_EGRID, 8, cout), jnp.float32),
        ],
    )(h, aux, part_in, pcur, wt, pnext)


def _final_body(r2, c3, k, h_ref, topd_ref, part_in_ref, pcur_ref, out_ref):
    mean, var = _reduce_stats(part_in_ref)
    gamma = pcur_ref[1:2, :].reshape(1, 1, c3)
    beta = pcur_ref[2:3, :].reshape(1, 1, c3)
    mean3 = mean.reshape(1, 1, c3)
    scale3 = lax.rsqrt(var + _EPS).reshape(1, 1, c3)
    h = h_ref[...]
    hn = gamma * (h - mean3) * scale3 + beta
    hn = jnp.maximum(hn, 0.0)
    valid = topd_ref[:, 0:k] <= r2
    acc = jnp.full((_SB, c3), -jnp.inf, jnp.float32)
    for j in range(k):
        vj = valid[:, j:j + 1]
        acc = jnp.maximum(acc, jnp.where(vj, hn[:, j, :], -jnp.inf))
    anyv = jnp.max(valid.astype(jnp.float32), axis=1, keepdims=True) > 0.0
    out_ref[...] = jnp.where(anyv, acc, 0.0)


def _run_final(r2, c3, k, h3d, topd, part_in, pcur):
    return pl.pallas_call(
        functools.partial(_final_body, r2, c3, k),
        grid=(_GRID,),
        in_specs=[
            pl.BlockSpec((_SB, k, c3), lambda i: (i, 0, 0)),
            pl.BlockSpec((_SB, _K), lambda i: (i, 0)),
            pl.BlockSpec((_EGRID, 8, c3), lambda i: (0, 0, 0)),
            pl.BlockSpec((8, c3), lambda i: (0, 0)),
        ],
        out_specs=pl.BlockSpec((_SB, c3), lambda i: (i, 0)),
        out_shape=jax.ShapeDtypeStruct((_S, c3), jnp.float32),
    )(h3d, topd, part_in, pcur)


def _pack_params(lyr, c):
    return jnp.concatenate(
        [lyr["b"][None, :], lyr["gamma"][None, :], lyr["beta"][None, :],
         jnp.zeros((5, c), jnp.float32)], axis=0)


def kernel(x, pos, batch, idx, params):
    x = x.astype(jnp.float32)
    pos = pos.astype(jnp.float32)
    pos_s = pos[idx]
    batch_s = batch[idx]
    posT = jnp.full((3, _NPAD), 1e9, jnp.float32).at[:, :_N].set(pos.T)
    topi, topd = _run_topk(posT, pos_s)

    table = jnp.concatenate(
        [x, pos, jnp.zeros((_N, _DTAB - _INC - 3), jnp.float32)], axis=1)
    xg = _sc_gather(table, topi.reshape(_S * _K))
    xg3 = xg.reshape(_S, _K, _DTAB)

    outs = []
    for si in range(3):
        r2 = float(_RADII[si]) * float(_RADII[si])
        k = _KS[si]
        c1, c2, c3 = _DIMS[si]
        layers = params[si]
        e = _S * k
        xg_s = xg3[:, :k, :].reshape(e, _DTAB)
        aux = jnp.concatenate(
            [jnp.repeat(pos_s, k, axis=0), topd[:, :k].reshape(e, 1)], axis=1)
        w1 = layers[0]["W"]
        w1xt = w1[:, :_INC].T
        w1pt = w1[:, _INC:_INC + 3].T
        p1 = _pack_params(layers[0], c1)
        p2 = _pack_params(layers[1], c2)
        p3 = _pack_params(layers[2], c3)
        h1, part1 = _run_layer1(r2, c1, k, xg_s, aux, w1xt, w1pt, p1)
        h2, part2 = _run_mid(r2, c1, c2, k, h1, aux, part1, p1,
                             layers[1]["W"].T, p2)
        h3, part3 = _run_mid(r2, c2, c3, k, h2, aux, part2, p2,
                             layers[2]["W"].T, p3)
        agg = _run_final(r2, c3, k, h3.reshape(_S, k, c3), topd, part3, p3)
        outs.append(agg)

    x_out = jnp.concatenate(outs, axis=1)
    return x_out, pos_s, batch_s
